# baseline (speedup 1.0000x reference)
import functools

import jax
import jax.numpy as jnp
from jax import lax
from jax.experimental import pallas as pl
from jax.experimental.pallas import tpu as pltpu
from jax.experimental.pallas import tpu_sc as plsc

KNBR = 16
EPS = 1e-5


def _lrelu(x):
    return jnp.where(x >= 0, x, 0.2 * x)


def _knn_body(cT_ref, c_ref, inds_ref):
    cT = cT_ref[...]
    c = c_ref[...]
    bb = jnp.sum(cT * cT, axis=1, keepdims=True)
    aa = jnp.sum(c * c, axis=0, keepdims=True)
    ab = jnp.dot(cT.astype(jnp.bfloat16), c.astype(jnp.bfloat16),
                 preferred_element_type=jnp.float32)
    d = aa - 2.0 * ab + bb
    n, rb = d.shape
    cand = lax.broadcasted_iota(jnp.int32, (n, rb), 0)
    candk = lax.broadcasted_iota(jnp.int32, (KNBR, rb), 0)
    outc = jnp.zeros((KNBR, rb), jnp.int32)
    for t in range(KNBR + 1):
        mn = jnp.min(d, axis=0, keepdims=True)
        idx = jnp.min(jnp.where(d == mn, cand, n), axis=0, keepdims=True)
        if t > 0:
            outc = jnp.where(candk == (t - 1), idx, outc)
        d = jnp.where(cand == idx, jnp.inf, d)
    inds_ref[...] = outc


def _mm1_body(fT_ref, u_ref, v_ref, p_ref, q_ref):
    fT = fT_ref[...]
    p_ref[...] = jnp.dot(fT, u_ref[...], preferred_element_type=jnp.float32)
    q_ref[...] = jnp.dot(fT, v_ref[...], preferred_element_type=jnp.float32)


def _norm_feats(P, g, d):
    gmax, gsum, gsq = g[:, :d], g[:, d:2 * d], g[:, 2 * d:]
    n = P.shape[0]
    cnt = jnp.float32(n * KNBR)
    sumx = KNBR * jnp.sum(P, 0, keepdims=True) + jnp.sum(gsum, 0, keepdims=True)
    sumx2 = (KNBR * jnp.sum(P * P, 0, keepdims=True)
             + 2.0 * jnp.sum(P * gsum, 0, keepdims=True)
             + jnp.sum(gsq, 0, keepdims=True))
    m = sumx / cnt
    v = sumx2 / cnt - m * m
    return _lrelu((P + gmax - m) / jnp.sqrt(v + EPS))


def _fin1_body(p_ref, g_ref, u_ref, v_ref, f1_ref, p2_ref, q2_ref):
    d = p_ref.shape[1]
    f = _norm_feats(p_ref[...], g_ref[...], d)
    f1_ref[...] = f
    p2_ref[...] = jnp.dot(f, u_ref[...], preferred_element_type=jnp.float32)
    q2_ref[...] = jnp.dot(f, v_ref[...], preferred_element_type=jnp.float32)


def _fin2_body(fT_ref, f1_ref, p2_ref, g_ref, t3f_ref, t3f1_ref, t3f2_ref,
               out_ref):
    d = p2_ref.shape[1]
    f2 = _norm_feats(p2_ref[...], g_ref[...], d)
    u = (jnp.dot(fT_ref[...], t3f_ref[...], preferred_element_type=jnp.float32)
         + jnp.dot(f1_ref[...], t3f1_ref[...],
                   preferred_element_type=jnp.float32)
         + jnp.dot(f2, t3f2_ref[...], preferred_element_type=jnp.float32))
    m3 = jnp.mean(u, 0, keepdims=True)
    v3 = jnp.mean(u * u, 0, keepdims=True) - m3 * m3
    out_ref[...] = _lrelu((u - m3) / jnp.sqrt(v3 + EPS))


def _tree(vals, op):
    while len(vals) > 1:
        half = len(vals) // 2
        vals = [op(vals[i], vals[i + half]) for i in range(half)]
    return vals[0]


def _make_gather(n, d):
    info = plsc.get_sparse_core_info()
    nw = info.num_cores * info.num_subcores
    npts = n // nw
    cp = 8 if d <= 256 else 4
    nchunks = npts // cp
    nidx = cp * KNBR
    nvr = nidx // 16
    mesh = plsc.VectorSubcoreMesh(core_axis_name="c", subcore_axis_name="s")

    @functools.partial(
        pl.kernel, mesh=mesh,
        out_type=jax.ShapeDtypeStruct((n, 3 * d), jnp.float32),
        scratch_types=[
            pltpu.VMEM((npts * KNBR,), jnp.int32),
            pltpu.VMEM((nchunks, nidx), jnp.int32),
            pltpu.VMEM((nidx, d), jnp.float32),
            pltpu.VMEM((nidx, d), jnp.float32),
            pltpu.VMEM((cp, 3 * d), jnp.float32),
            pltpu.VMEM((cp, 3 * d), jnp.float32),
            pltpu.SemaphoreType.DMA,
            pltpu.SemaphoreType.DMA,
            pltpu.SemaphoreType.DMA,
            pltpu.SemaphoreType.DMA,
        ],
    )
    def gather_kernel(qt_hbm, idx_hbm, out_hbm,
                      idxf_v, idxc_v, rows0_v, rows1_v, ob0_v, ob1_v,
                      gs0, gs1, os0, os1):
        wid = lax.axis_index("s") * info.num_cores + lax.axis_index("c")
        rows_b = (rows0_v, rows1_v)
        ob_b = (ob0_v, ob1_v)
        gs_b = (gs0, gs1)
        os_b = (os0, os1)

        pltpu.sync_copy(idx_hbm.at[pl.ds(wid * npts * KNBR, npts * KNBR)],
                        idxf_v)
        for u in range(npts * KNBR // 16):
            ci, col = divmod(u, nvr)
            idxc_v[ci, pl.ds(col * 16, 16)] = idxf_v[pl.ds(u * 16, 16)]

        def fetch(ci, s):
            pltpu.async_copy(qt_hbm.at[idxc_v.at[ci]], rows_b[s], gs_b[s])

        fetch(0, 0)
        fetch(1, 1)

        def halfstep(ci, s):
            rows_v, ob_v = rows_b[s], ob_b[s]
            pltpu.make_async_copy(qt_hbm.at[idxc_v.at[0]], rows_v,
                                  gs_b[s]).wait()

            @pl.when(ci >= 2)
            def _():
                pltpu.make_async_copy(
                    ob_v, out_hbm.at[pl.ds(wid * npts, cp)], os_b[s]).wait()

            def pt_body(p, c2):
                add = lambda a, b: a + b
                for j in range(d // 16):
                    sl = pl.ds(j * 16, 16)
                    mxs, sms, sqs = [], [], []
                    for g0 in range(0, KNBR, 4):
                        v0 = rows_v[p * KNBR + g0, sl]
                        mx, sm, sq = v0, v0, v0 * v0
                        for r in range(g0 + 1, g0 + 4):
                            vr = rows_v[p * KNBR + r, sl]
                            mx = jnp.maximum(mx, vr)
                            sm = sm + vr
                            sq = sq + vr * vr
                        mxs.append(mx)
                        sms.append(sm)
                        sqs.append(sq)
                    ob_v[p, pl.ds(j * 16, 16)] = _tree(mxs, jnp.maximum)
                    ob_v[p, pl.ds(d + j * 16, 16)] = _tree(sms, add)
                    ob_v[p, pl.ds(2 * d + j * 16, 16)] = _tree(sqs, add)
                return c2

            lax.fori_loop(0, cp, pt_body, 0)
            pltpu.async_copy(ob_v, out_hbm.at[pl.ds(wid * npts + ci * cp, cp)],
                             os_b[s])

            @pl.when(ci + 2 < nchunks)
            def _():
                fetch(ci + 2, s)

        def pair_body(c2, carry):
            halfstep(2 * c2, 0)
            halfstep(2 * c2 + 1, 1)
            return carry

        lax.fori_loop(0, nchunks // 2, pair_body, 0)
        for s in range(2):
            pltpu.make_async_copy(
                ob_b[s], out_hbm.at[pl.ds(wid * npts, cp)], os_b[s]).wait()

    return gather_kernel


def kernel(coords, feats, W1, W2, W3):
    B, C, N = feats.shape
    cT = coords.transpose(0, 2, 1)
    fT = feats.transpose(0, 2, 1)
    U1, V1 = (W1[:, :C] - W1[:, C:]).T, W1[:, C:].T
    U2, V2 = (W2[:, :C] - W2[:, C:]).T, W2[:, C:].T
    T3f = W3[:, :C].T
    T3f1 = W3[:, C:2 * C].T
    T3f2 = W3[:, 2 * C:].T
    f32 = jnp.float32

    RB = 512
    knn = pl.pallas_call(
        _knn_body,
        grid=(N // RB,),
        in_specs=[pl.BlockSpec((N, 3), lambda i: (0, 0)),
                  pl.BlockSpec((3, RB), lambda i: (0, i))],
        out_specs=pl.BlockSpec((KNBR, RB), lambda i: (0, i)),
        out_shape=jax.ShapeDtypeStruct((KNBR, N), jnp.int32),
    )
    full = lambda r, c: pl.BlockSpec((r, c), lambda: (0, 0))
    mm1 = pl.pallas_call(
        _mm1_body,
        in_specs=[full(N, C), full(C, C), full(C, C)],
        out_specs=[full(N, C)] * 2,
        out_shape=[jax.ShapeDtypeStruct((N, C), f32)] * 2,
    )
    fin1 = pl.pallas_call(
        _fin1_body,
        in_specs=[full(N, C), full(N, 3 * C), full(C, 2 * C), full(C, 2 * C)],
        out_specs=[full(N, C), full(N, 2 * C), full(N, 2 * C)],
        out_shape=[jax.ShapeDtypeStruct((N, C), f32),
                   jax.ShapeDtypeStruct((N, 2 * C), f32),
                   jax.ShapeDtypeStruct((N, 2 * C), f32)],
    )
    fin2 = pl.pallas_call(
        _fin2_body,
        in_specs=[full(N, C), full(N, C), full(N, 2 * C), full(N, 6 * C),
                  full(C, C), full(C, C), full(2 * C, C)],
        out_specs=full(N, C),
        out_shape=jax.ShapeDtypeStruct((N, C), f32),
    )
    gather1 = _make_gather(N, C)
    gather2 = _make_gather(N, 2 * C)

    outs = []
    for b in range(B):
        inds = knn(cT[b], coords[b])
        idx_flat = inds.T.reshape(N * KNBR)
        P1T, Q1T = mm1(fT[b], U1, V1)
        g1 = gather1(Q1T, idx_flat)
        f1T, P2T, Q2T = fin1(P1T, g1, U2, V2)
        g2 = gather2(Q2T, idx_flat)
        outs.append(fin2(fT[b], f1T, P2T, g2, T3f, T3f1, T3f2))

    return jnp.stack(outs).transpose(0, 2, 1)


# speedup vs baseline: 1.0520x; 1.0520x over previous
import functools

import jax
import jax.numpy as jnp
from jax import lax
from jax.experimental import pallas as pl
from jax.experimental.pallas import tpu as pltpu
from jax.experimental.pallas import tpu_sc as plsc

KNBR = 16
EPS = 1e-5


def _lrelu(x):
    return jnp.where(x >= 0, x, 0.2 * x)


def _knn_body(cT_ref, c_ref, inds_ref):
    cT = cT_ref[...]
    c = c_ref[0]
    a2c = jnp.sum(cT * cT, axis=1, keepdims=True)
    a2r = jnp.sum(c * c, axis=0, keepdims=True)
    ab = jnp.dot(cT.astype(jnp.bfloat16), c.astype(jnp.bfloat16),
                 preferred_element_type=jnp.float32)
    d = a2c - 2.0 * ab + a2r
    rb, n = d.shape
    lane = lax.broadcasted_iota(jnp.int32, (rb, n), 1)
    lanek = lax.broadcasted_iota(jnp.int32, (rb, KNBR), 1)
    outc = jnp.zeros((rb, KNBR), jnp.int32)
    for t in range(KNBR + 1):
        mn = jnp.min(d, axis=1, keepdims=True)
        idx = jnp.min(jnp.where(d == mn, lane, n), axis=1, keepdims=True)
        if t > 0:
            outc = jnp.where(lanek == (t - 1), idx, outc)
        d = jnp.where(lane == idx, jnp.inf, d)
    inds_ref[...] = outc


def _mm1_body(fT_ref, u_ref, v_ref, p_ref, q_ref):
    fT = fT_ref[...]
    p_ref[...] = jnp.dot(fT, u_ref[...], preferred_element_type=jnp.float32)
    q_ref[...] = jnp.dot(fT, v_ref[...], preferred_element_type=jnp.float32)


def _norm_feats(P, g, d):
    gmax, gsum, gsq = g[:, :d], g[:, d:2 * d], g[:, 2 * d:]
    n = P.shape[0]
    cnt = jnp.float32(n * KNBR)
    sumx = KNBR * jnp.sum(P, 0, keepdims=True) + jnp.sum(gsum, 0, keepdims=True)
    sumx2 = (KNBR * jnp.sum(P * P, 0, keepdims=True)
             + 2.0 * jnp.sum(P * gsum, 0, keepdims=True)
             + jnp.sum(gsq, 0, keepdims=True))
    m = sumx / cnt
    v = sumx2 / cnt - m * m
    return _lrelu((P + gmax - m) / jnp.sqrt(v + EPS))


def _fin1_body(p_ref, g_ref, u_ref, v_ref, f1_ref, p2_ref, q2_ref):
    d = p_ref.shape[1]
    f = _norm_feats(p_ref[...], g_ref[...], d)
    f1_ref[...] = f
    p2_ref[...] = jnp.dot(f, u_ref[...], preferred_element_type=jnp.float32)
    q2_ref[...] = jnp.dot(
        f, v_ref[...], preferred_element_type=jnp.float32).astype(q2_ref.dtype)


def _fin2_body(fT_ref, f1_ref, p2_ref, g_ref, t3f_ref, t3f1_ref, t3f2_ref,
               out_ref):
    d = p2_ref.shape[1]
    g = g_ref[...].astype(jnp.float32)
    f2 = _norm_feats(p2_ref[...], g, d)
    u = (jnp.dot(fT_ref[...], t3f_ref[...], preferred_element_type=jnp.float32)
         + jnp.dot(f1_ref[...], t3f1_ref[...],
                   preferred_element_type=jnp.float32)
         + jnp.dot(f2, t3f2_ref[...], preferred_element_type=jnp.float32))
    m3 = jnp.mean(u, 0, keepdims=True)
    v3 = jnp.mean(u * u, 0, keepdims=True) - m3 * m3
    out_ref[...] = _lrelu((u - m3) / jnp.sqrt(v3 + EPS))


def _tree(vals, op):
    while len(vals) > 1:
        half = len(vals) // 2
        vals = [op(vals[i], vals[i + half]) for i in range(half)]
    return vals[0]


def _make_gather(n, nb, d, dtype):
    info = plsc.get_sparse_core_info()
    nw = info.num_cores * info.num_subcores
    npts = (nb * n) // nw
    esz = jnp.dtype(dtype).itemsize
    cp = 8 if d * esz <= 1024 else 4
    lw = 16 * (4 // esz)
    nchunks = npts // cp
    nidx = cp * KNBR
    nvr = nidx // 16
    wpb = nw // nb
    mesh = plsc.VectorSubcoreMesh(core_axis_name="c", subcore_axis_name="s")

    @functools.partial(
        pl.kernel, mesh=mesh,
        out_type=jax.ShapeDtypeStruct((nb * n, 3 * d), dtype),
        scratch_types=[
            pltpu.VMEM((npts * KNBR,), jnp.int32),
            pltpu.VMEM((nchunks, nidx), jnp.int32),
            pltpu.VMEM((nidx, d), dtype),
            pltpu.VMEM((nidx, d), dtype),
            pltpu.VMEM((cp, 3 * d), dtype),
            pltpu.VMEM((cp, 3 * d), dtype),
            pltpu.SemaphoreType.DMA,
            pltpu.SemaphoreType.DMA,
            pltpu.SemaphoreType.DMA,
            pltpu.SemaphoreType.DMA,
        ],
    )
    def gather_kernel(qt_hbm, idx_hbm, out_hbm,
                      idxf_v, idxc_v, rows0_v, rows1_v, ob0_v, ob1_v,
                      gs0, gs1, os0, os1):
        wid = lax.axis_index("s") * info.num_cores + lax.axis_index("c")
        boff = (wid // wpb) * n
        rows_b = (rows0_v, rows1_v)
        ob_b = (ob0_v, ob1_v)
        gs_b = (gs0, gs1)
        os_b = (os0, os1)

        pltpu.sync_copy(idx_hbm.at[pl.ds(wid * npts * KNBR, npts * KNBR)],
                        idxf_v)
        for u in range(npts * KNBR // 16):
            ci, col = divmod(u, nvr)
            v = idxf_v[pl.ds(u * 16, 16)]
            idxc_v[ci, pl.ds(col * 16, 16)] = v + boff if nb > 1 else v

        def fetch(ci, s):
            pltpu.async_copy(qt_hbm.at[idxc_v.at[ci]], rows_b[s], gs_b[s])

        fetch(0, 0)
        fetch(1, 1)

        def halfstep(ci, s):
            rows_v, ob_v = rows_b[s], ob_b[s]
            pltpu.make_async_copy(qt_hbm.at[idxc_v.at[0]], rows_v,
                                  gs_b[s]).wait()

            @pl.when(ci >= 2)
            def _():
                pltpu.make_async_copy(
                    ob_v, out_hbm.at[pl.ds(wid * npts, cp)], os_b[s]).wait()

            add = lambda a, b: a + b

            def reduce_point(p, j):
                sl = pl.ds(pl.multiple_of(j * lw, lw), lw)
                mxs, sms, sqs = [], [], []
                for g0 in range(0, KNBR, 4):
                    v0 = rows_v[p * KNBR + g0, sl]
                    mx, sm, sq = v0, v0, v0 * v0
                    for r in range(g0 + 1, g0 + 4):
                        vr = rows_v[p * KNBR + r, sl]
                        mx = jnp.maximum(mx, vr)
                        sm = sm + vr
                        sq = sq + vr * vr
                    mxs.append(mx)
                    sms.append(sm)
                    sqs.append(sq)
                ob_v[p, pl.ds(pl.multiple_of(j * lw, lw), lw)] = _tree(
                    mxs, jnp.maximum)
                ob_v[p, pl.ds(pl.multiple_of(d + j * lw, lw), lw)] = _tree(
                    sms, add)
                ob_v[p, pl.ds(pl.multiple_of(2 * d + j * lw, lw), lw)] = _tree(
                    sqs, add)

            if esz == 4:
                def pt_body(p, c2):
                    for j in range(d // lw):
                        reduce_point(p, j)
                    return c2

                lax.fori_loop(0, cp, pt_body, 0)
            else:
                def col_body(j, c2):
                    for p in range(cp):
                        reduce_point(p, j)
                    return c2

                lax.fori_loop(0, d // lw, col_body, 0)
            pltpu.async_copy(ob_v, out_hbm.at[pl.ds(wid * npts + ci * cp, cp)],
                             os_b[s])

            @pl.when(ci + 2 < nchunks)
            def _():
                fetch(ci + 2, s)

        def pair_body(c2, carry):
            halfstep(2 * c2, 0)
            halfstep(2 * c2 + 1, 1)
            return carry

        lax.fori_loop(0, nchunks // 2, pair_body, 0)
        for s in range(2):
            pltpu.make_async_copy(
                ob_b[s], out_hbm.at[pl.ds(wid * npts, cp)], os_b[s]).wait()

    return gather_kernel


def kernel(coords, feats, W1, W2, W3):
    B, C, N = feats.shape
    cT = coords.transpose(0, 2, 1)
    fT = feats.transpose(0, 2, 1)
    U1, V1 = (W1[:, :C] - W1[:, C:]).T, W1[:, C:].T
    U2, V2 = (W2[:, :C] - W2[:, C:]).T, W2[:, C:].T
    T3f = W3[:, :C].T
    T3f1 = W3[:, C:2 * C].T
    T3f2 = W3[:, 2 * C:].T
    f32 = jnp.float32

    fT2 = fT.reshape(B * N, C)
    cT2 = cT.reshape(B * N, 3)

    RB = 512
    nrb = N // RB
    inds = pl.pallas_call(
        _knn_body,
        grid=(B, nrb),
        in_specs=[pl.BlockSpec((RB, 3), lambda b, i: (b * nrb + i, 0)),
                  pl.BlockSpec((1, 3, N), lambda b, i: (b, 0, 0))],
        out_specs=pl.BlockSpec((RB, KNBR), lambda b, i: (b * nrb + i, 0)),
        out_shape=jax.ShapeDtypeStruct((B * N, KNBR), jnp.int32),
    )(cT2, coords)
    idx_flat = inds.reshape(B * N * KNBR)

    bspec = lambda dd: pl.BlockSpec((N, dd), lambda b: (b, 0))
    wspec = lambda r, c: pl.BlockSpec((r, c), lambda b: (0, 0))

    P1T, Q1T = pl.pallas_call(
        _mm1_body,
        grid=(B,),
        in_specs=[bspec(C), wspec(C, C), wspec(C, C)],
        out_specs=[bspec(C)] * 2,
        out_shape=[jax.ShapeDtypeStruct((B * N, C), f32)] * 2,
    )(fT2, U1, V1)

    g1 = _make_gather(N, B, C, f32)(Q1T, idx_flat)

    f1T, P2T, Q2T = pl.pallas_call(
        _fin1_body,
        grid=(B,),
        in_specs=[bspec(C), bspec(3 * C), wspec(C, 2 * C), wspec(C, 2 * C)],
        out_specs=[bspec(C), bspec(2 * C), bspec(2 * C)],
        out_shape=[jax.ShapeDtypeStruct((B * N, C), f32),
                   jax.ShapeDtypeStruct((B * N, 2 * C), f32),
                   jax.ShapeDtypeStruct((B * N, 2 * C), f32)],
    )(P1T, g1, U2, V2)

    g2 = _make_gather(N, B, 2 * C, f32)(Q2T, idx_flat)

    outT = pl.pallas_call(
        _fin2_body,
        grid=(B,),
        in_specs=[bspec(C), bspec(C), bspec(2 * C), bspec(6 * C),
                  wspec(C, C), wspec(C, C), wspec(2 * C, C)],
        out_specs=bspec(C),
        out_shape=jax.ShapeDtypeStruct((B * N, C), f32),
    )(fT2, f1T, P2T, g2, T3f, T3f1, T3f2)

    return outT.reshape(B, N, C).transpose(0, 2, 1)
